# baseline (speedup 1.0000x reference)
import functools

import jax
import jax.numpy as jnp
from jax import lax
from jax.experimental import pallas as pl
from jax.experimental.pallas import tpu as pltpu
from jax.experimental.pallas import tpu_sc as plsc

_NC, _NS, _LANES = 2, 16, 16
_NW = _NC * _NS
_CHUNK = 128
_RB = 32768


_G = 256


def _dense_body(mt_ref, p_ref, o_ref):
    d = p_ref.shape[0]
    g = _G
    nb = o_ref.shape[1] // (2 * g)
    pi = lax.broadcasted_iota(jnp.int32, (g, 2 * g), 0)
    ci = lax.broadcasted_iota(jnp.int32, (g, 2 * g), 1)
    ev = (ci == 2 * pi).astype(jnp.float32)
    od = (ci == 2 * pi + 1).astype(jnp.float32)
    dn = (((0,), (0,)), ((), ()))
    for k in range(nb):
        chunk = mt_ref[pl.ds(k * g, g), :]
        a = lax.dot_general(chunk[:, :d], ev, dn,
                            preferred_element_type=jnp.float32)
        b = lax.dot_general(chunk[:, d:], od, dn,
                            preferred_element_type=jnp.float32)
        sl = pl.ds(k * 2 * g, 2 * g)
        o_ref[:, sl] = p_ref[:, sl] + 0.5 * (a + b)


def kernel(mem, idx, val, pretrained):
    M, D = mem.shape
    B = idx.shape[0]

    n_chunks = B // _CHUNK
    cpw = n_chunks // _NW
    idx2d = idx.astype(jnp.int32).reshape(n_chunks, _CHUNK)
    nvec = D // _LANES

    mesh = plsc.VectorSubcoreMesh(
        core_axis_name="c", subcore_axis_name="s",
        num_cores=_NC, num_subcores=_NS,
    )

    @functools.partial(
        pl.kernel,
        out_type=(),
        mesh=mesh,
        compiler_params=pltpu.CompilerParams(use_tc_tiling_on_sc=False),
        scratch_types=[
            pltpu.VMEM((cpw, _CHUNK), jnp.int32),
            pltpu.VMEM((_CHUNK, D), jnp.float32),
            pltpu.VMEM((_CHUNK, D), jnp.float32),
            pltpu.SemaphoreType.DMA,
        ],
    )
    def sc_merge(idx_hbm, val_hbm, mem_ref, idx_v, cur_v, val_v, sem):
        wid = lax.axis_index("s") * _NC + lax.axis_index("c")
        pltpu.sync_copy(idx_hbm.at[pl.ds(wid * cpw, cpw)], idx_v)
        for j in range(cpw):
            idx_row = idx_v.at[j]
            pltpu.async_copy(mem_ref.at[idx_row], cur_v, sem).wait()
            row0 = (wid * cpw + j) * _CHUNK
            pltpu.sync_copy(val_hbm.at[pl.ds(row0, _CHUNK)], val_v)

            def row_body(r, acc):
                for c in range(nvec):
                    sl = pl.ds(c * _LANES, _LANES)
                    cu = cur_v[r, sl]
                    va = val_v[r, sl]
                    cur_v[r, sl] = jnp.where(jnp.abs(va) > jnp.abs(cu), va, cu)
                return acc

            lax.fori_loop(0, _CHUNK, row_body, 0)
            pltpu.async_copy(cur_v, mem_ref.at[idx_row], sem).wait()

    mref = jax.new_ref(mem)
    sc_merge(idx2d, val, mref)
    merged = mref[...]

    rb = _RB
    merged2 = merged.reshape(M // 2, 2 * D)
    out_t = pl.pallas_call(
        _dense_body,
        grid=(pl.cdiv(M, rb),),
        in_specs=[
            pl.BlockSpec((rb // 2, 2 * D), lambda i: (i, 0)),
            pl.BlockSpec((D, rb), lambda i: (0, i)),
        ],
        out_specs=pl.BlockSpec((D, rb), lambda i: (0, i)),
        out_shape=jax.ShapeDtypeStruct((D, M), jnp.float32),
    )(merged2, pretrained.T)
    return out_t.T


# speedup vs baseline: 1.0179x; 1.0179x over previous
import functools

import jax
import jax.numpy as jnp
from jax import lax
from jax.experimental import pallas as pl
from jax.experimental.pallas import tpu as pltpu
from jax.experimental.pallas import tpu_sc as plsc

_NC, _NS, _LANES = 2, 16, 16
_NW = _NC * _NS
_CHUNK = 128
_RB = 32768


_G = 128


def _dense_body(mt_ref, p_ref, o_ref):
    d = p_ref.shape[0]
    g = _G
    nb = o_ref.shape[1] // (2 * g)
    pi = lax.broadcasted_iota(jnp.int32, (g, 2 * g), 0)
    ci = lax.broadcasted_iota(jnp.int32, (g, 2 * g), 1)
    ev = (ci == 2 * pi).astype(jnp.float32)
    od = (ci == 2 * pi + 1).astype(jnp.float32)
    dn = (((0,), (0,)), ((), ()))
    for k in range(nb):
        chunk = mt_ref[pl.ds(k * g, g), :]
        a = lax.dot_general(chunk[:, :d], ev, dn,
                            preferred_element_type=jnp.float32)
        b = lax.dot_general(chunk[:, d:], od, dn,
                            preferred_element_type=jnp.float32)
        sl = pl.ds(k * 2 * g, 2 * g)
        o_ref[:, sl] = p_ref[:, sl] + 0.5 * (a + b)


def kernel(mem, idx, val, pretrained):
    M, D = mem.shape
    B = idx.shape[0]

    n_chunks = B // _CHUNK
    cpw = n_chunks // _NW
    idx2d = idx.astype(jnp.int32).reshape(n_chunks, _CHUNK)
    nvec = D // _LANES

    mesh = plsc.VectorSubcoreMesh(
        core_axis_name="c", subcore_axis_name="s",
        num_cores=_NC, num_subcores=_NS,
    )

    @functools.partial(
        pl.kernel,
        out_type=(),
        mesh=mesh,
        compiler_params=pltpu.CompilerParams(use_tc_tiling_on_sc=False),
        scratch_types=[
            pltpu.VMEM((cpw, _CHUNK), jnp.int32),
            pltpu.VMEM((_CHUNK, D), jnp.float32),
            pltpu.VMEM((_CHUNK, D), jnp.float32),
            pltpu.SemaphoreType.DMA,
        ],
    )
    def sc_merge(idx_hbm, val_hbm, mem_ref, idx_v, cur_v, val_v, sem):
        wid = lax.axis_index("s") * _NC + lax.axis_index("c")
        pltpu.sync_copy(idx_hbm.at[pl.ds(wid * cpw, cpw)], idx_v)
        for j in range(cpw):
            idx_row = idx_v.at[j]
            pltpu.async_copy(mem_ref.at[idx_row], cur_v, sem).wait()
            row0 = (wid * cpw + j) * _CHUNK
            pltpu.sync_copy(val_hbm.at[pl.ds(row0, _CHUNK)], val_v)

            def row_body(r, acc):
                for c in range(nvec):
                    sl = pl.ds(c * _LANES, _LANES)
                    cu = cur_v[r, sl]
                    va = val_v[r, sl]
                    cur_v[r, sl] = jnp.where(jnp.abs(va) > jnp.abs(cu), va, cu)
                return acc

            lax.fori_loop(0, _CHUNK, row_body, 0)
            pltpu.async_copy(cur_v, mem_ref.at[idx_row], sem).wait()

    mref = jax.new_ref(mem)
    sc_merge(idx2d, val, mref)
    merged = mref[...]

    rb = _RB
    merged2 = merged.reshape(M // 2, 2 * D)
    out_t = pl.pallas_call(
        _dense_body,
        grid=(pl.cdiv(M, rb),),
        in_specs=[
            pl.BlockSpec((rb // 2, 2 * D), lambda i: (i, 0)),
            pl.BlockSpec((D, rb), lambda i: (0, i)),
        ],
        out_specs=pl.BlockSpec((D, rb), lambda i: (0, i)),
        out_shape=jax.ShapeDtypeStruct((D, M), jnp.float32),
    )(merged2, pretrained.T)
    return out_t.T
